# baseline (speedup 1.0000x reference)
import jax
import jax.numpy as jnp
from jax.experimental import pallas as pl
from jax.experimental.pallas import tpu as pltpu
from jax.experimental.pallas import tpu_sc as plsc

T = 2048
D = 768
E = 8
S = 128
H = 128
CHUNK = 512
NCHUNK = T // CHUNK
LANES = 16
NSUB = S // LANES
SCCHUNK = 1024
NSCCHUNK = T // SCCHUNK


def _routes_from_tokens(tid):
    x = tid
    x = x ^ jax.lax.shift_right_logical(x, 16)
    x = x * jnp.int32(-2048144789)
    x = x ^ jax.lax.shift_right_logical(x, 13)
    x = x * jnp.int32(-1028477387)
    x = x ^ jax.lax.shift_right_logical(x, 16)
    return x & jnp.int32(E - 1)


def _gates_kernel(x_ref, x0_ref, tid_ref, win_ref, wsi_ref, wso_ref, dp_ref,
                  rm_ref, xm_ref, r_ref, a_ref, bu_ref, c_ref, dd_ref):
    rm = rm_ref[...]
    xm = rm[0:1, :] * x_ref[...] + rm[1:2, :] * x0_ref[...]
    xm_ref[...] = xm
    ms = jnp.mean(xm * xm, axis=1, keepdims=True)
    xn = xm * jax.lax.rsqrt(ms + 1e-6)

    r = _routes_from_tokens(tid_ref[...])
    r_ref[...] = r

    f32 = jnp.float32
    bf16 = jnp.bfloat16
    xnb = xn.astype(bf16)
    u = jnp.zeros((CHUNK, S), f32)
    selz = jnp.zeros((CHUNK, H), f32)
    for e in range(E):
        xe = jnp.where(r == e, xnb, bf16(0))
        u = u + jnp.dot(xe, win_ref[e], preferred_element_type=f32)
        selz = selz + jnp.dot(xe, wsi_ref[e], preferred_element_type=f32)
    sel = (selz * jax.nn.sigmoid(selz)).astype(bf16)
    so = jnp.zeros((CHUNK, 4 * S), f32)
    dp = jnp.zeros((CHUNK, S), f32)
    for e in range(E):
        se = jnp.where(r == e, sel, bf16(0))
        so = so + jnp.dot(se, wso_ref[e], preferred_element_type=f32)
        dp = dp + (r == e).astype(f32) * dp_ref[e:e + 1, :]
    a = jax.nn.sigmoid(so[:, 0:S])
    b = jnp.tanh(so[:, S:2 * S])
    c = jnp.tanh(so[:, 2 * S:3 * S])
    dg = jax.nn.sigmoid(so[:, 3 * S:4 * S])
    a_ref[...] = a
    bu_ref[...] = b * u
    c_ref[...] = c
    dd_ref[...] = dp * dg * u


def _scan_kernel(r_ref, a_ref, bu_ref, c_ref, dd_ref, y_ref):
    eidx = jax.lax.broadcasted_iota(jnp.int32, (E, 1), 0)

    def body(blk, h):
        t0 = pl.multiple_of(blk * 8, 8)
        a8 = a_ref[pl.ds(t0, 8), :]
        bu8 = bu_ref[pl.ds(t0, 8), :]
        c8 = c_ref[pl.ds(t0, 8), :]
        dd8 = dd_ref[pl.ds(t0, 8), :]
        y8 = dd8
        for j in range(8):
            rt = r_ref[t0 + j]
            mask = eidx == rt
            aj = a8[j:j + 1, :]
            buj = bu8[j:j + 1, :]
            h = jnp.where(mask, aj * h + buj, h)
            hr = jnp.sum(jnp.where(mask, h, 0.0), axis=0, keepdims=True)
            yj = c8[j:j + 1, :] * hr
            y8 = y8 + jnp.where(jax.lax.broadcasted_iota(jnp.int32, (8, 1), 0) == j,
                                yj, 0.0)
        y_ref[pl.ds(t0, 8), :] = y8
        return h

    jax.lax.fori_loop(0, T // 8, body, jnp.zeros((E, S), jnp.float32))


def _out_kernel(y_ref, r_ref, xm_ref, wout_ref, ssm_ref, mlp_ref,
                w1_ref, w2_ref, o_ref):
    f32 = jnp.float32
    bf16 = jnp.bfloat16
    r = r_ref[...]
    y = y_ref[...].astype(bf16)
    out = jnp.zeros((CHUNK, D), f32)
    for e in range(E):
        ye = jnp.where(r == e, y, bf16(0))
        out = out + jnp.dot(ye, wout_ref[e], preferred_element_type=f32)
    xm2 = xm_ref[...] + ssm_ref[...] * out
    ms = jnp.mean(xm2 * xm2, axis=1, keepdims=True)
    xn2 = xm2 * jax.lax.rsqrt(ms + 1e-6)
    hmid = jnp.dot(xn2.astype(bf16), w1_ref[...], preferred_element_type=f32)
    hmid = jnp.square(jnp.maximum(hmid, 0.0)).astype(bf16)
    mlp = jnp.dot(hmid, w2_ref[...], preferred_element_type=f32)
    o_ref[...] = xm2 + mlp_ref[...] * mlp


def kernel(x, x0, token_ids, W_in, W_sel_in, W_sel_out, W_out, d_param,
           resid_mix, ssm_scale, mlp_scale, W_mlp1, W_mlp2):
    f32 = jnp.float32
    x2 = x.reshape(T, D)
    x02 = x0.reshape(T, D)
    tid = token_ids.reshape(T, 1)

    full = lambda shape: pl.BlockSpec(shape, lambda i: tuple(0 for _ in shape))
    chunk = lambda shape: pl.BlockSpec(shape, lambda i: (i,) + tuple(0 for _ in shape[1:]))

    xm, r, a, bu, c, dd = pl.pallas_call(
        _gates_kernel,
        grid=(NCHUNK,),
        in_specs=[
            chunk((CHUNK, D)), chunk((CHUNK, D)), chunk((CHUNK, 1)),
            full((E, D, S)), full((E, D, H)), full((E, H, 4 * S)),
            full((E, S)), full((2, D)),
        ],
        out_specs=[
            chunk((CHUNK, D)), chunk((CHUNK, 1)), chunk((CHUNK, S)),
            chunk((CHUNK, S)), chunk((CHUNK, S)), chunk((CHUNK, S)),
        ],
        out_shape=[
            jax.ShapeDtypeStruct((T, D), f32),
            jax.ShapeDtypeStruct((T, 1), jnp.int32),
            jax.ShapeDtypeStruct((T, S), f32),
            jax.ShapeDtypeStruct((T, S), f32),
            jax.ShapeDtypeStruct((T, S), f32),
            jax.ShapeDtypeStruct((T, S), f32),
        ],
    )(x2, x02, tid, W_in.astype(jnp.bfloat16), W_sel_in.astype(jnp.bfloat16),
      W_sel_out.astype(jnp.bfloat16), d_param, resid_mix)

    y = pl.pallas_call(
        _scan_kernel,
        grid_spec=pltpu.PrefetchScalarGridSpec(
            num_scalar_prefetch=1,
            grid=(1,),
            in_specs=[
                pl.BlockSpec((T, S), lambda i, s: (0, 0)),
                pl.BlockSpec((T, S), lambda i, s: (0, 0)),
                pl.BlockSpec((T, S), lambda i, s: (0, 0)),
                pl.BlockSpec((T, S), lambda i, s: (0, 0)),
            ],
            out_specs=pl.BlockSpec((T, S), lambda i, s: (0, 0)),
        ),
        out_shape=jax.ShapeDtypeStruct((T, S), f32),
    )(r.reshape(T), a, bu, c, dd)

    o = pl.pallas_call(
        _out_kernel,
        grid=(NCHUNK,),
        in_specs=[
            chunk((CHUNK, S)), chunk((CHUNK, 1)), chunk((CHUNK, D)),
            full((E, S, D)), full((1, D)), full((1, D)),
            full((D, 4 * D)), full((4 * D, D)),
        ],
        out_specs=chunk((CHUNK, D)),
        out_shape=jax.ShapeDtypeStruct((T, D), f32),
    )(y, r, xm, W_out.astype(jnp.bfloat16), ssm_scale.reshape(1, D),
      mlp_scale.reshape(1, D), W_mlp1.astype(jnp.bfloat16),
      W_mlp2.astype(jnp.bfloat16))

    return o.reshape(1, T, D)


# speedup vs baseline: 1.0054x; 1.0054x over previous
import jax
import jax.numpy as jnp
from jax.experimental import pallas as pl
from jax.experimental.pallas import tpu as pltpu
from jax.experimental.pallas import tpu_sc as plsc

T = 2048
D = 768
E = 8
S = 128
H = 128
CHUNK = 512
NCHUNK = T // CHUNK
LANES = 16
NSUB = S // LANES
SCCHUNK = 1024
NSCCHUNK = T // SCCHUNK


def _routes_from_tokens(tid):
    x = tid
    x = x ^ jax.lax.shift_right_logical(x, 16)
    x = x * jnp.int32(-2048144789)
    x = x ^ jax.lax.shift_right_logical(x, 13)
    x = x * jnp.int32(-1028477387)
    x = x ^ jax.lax.shift_right_logical(x, 16)
    return x & jnp.int32(E - 1)


def _gates_kernel(x_ref, x0_ref, tid_ref, win_ref, wsi_ref, wso_ref, dp_ref,
                  rm_ref, xm_ref, r_ref, y_ref, a_s, bu_s, c_s, dd_s, r_s):
    i = pl.program_id(0)
    rm = rm_ref[...]
    xm = rm[0:1, :] * x_ref[...] + rm[1:2, :] * x0_ref[...]
    xm_ref[...] = xm
    ms = jnp.mean(xm * xm, axis=1, keepdims=True)
    xn = xm * jax.lax.rsqrt(ms + 1e-6)

    r = _routes_from_tokens(tid_ref[...])
    r_ref[...] = r

    f32 = jnp.float32
    u = jnp.zeros((CHUNK, S), f32)
    selz = jnp.zeros((CHUNK, H), f32)
    for e in range(E):
        xe = jnp.where(r == e, xn, 0.0)
        u = u + jnp.dot(xe, win_ref[e], preferred_element_type=f32)
        selz = selz + jnp.dot(xe, wsi_ref[e], preferred_element_type=f32)
    sel = selz * jax.nn.sigmoid(selz)
    so = jnp.zeros((CHUNK, 4 * S), f32)
    dp = jnp.zeros((CHUNK, S), f32)
    for e in range(E):
        se = jnp.where(r == e, sel, 0.0)
        so = so + jnp.dot(se, wso_ref[e], preferred_element_type=f32)
        dp = dp + (r == e).astype(f32) * dp_ref[e:e + 1, :]
    a = jax.nn.sigmoid(so[:, 0:S])
    b = jnp.tanh(so[:, S:2 * S])
    c = jnp.tanh(so[:, 2 * S:3 * S])
    dg = jax.nn.sigmoid(so[:, 3 * S:4 * S])
    rows = pl.ds(pl.multiple_of(i * CHUNK, CHUNK), CHUNK)
    a_s[rows, :] = a
    bu_s[rows, :] = b * u
    c_s[rows, :] = c
    dd_s[rows, :] = dp * dg * u
    r_s[rows, :] = r

    @pl.when(i == NCHUNK - 1)
    def _scan():
        eidx = jax.lax.broadcasted_iota(jnp.int32, (E, 1), 0)
        sidx = jax.lax.broadcasted_iota(jnp.int32, (8, 1), 0)

        def body(blk, h):
            t0 = pl.multiple_of(blk * 8, 8)
            a8 = a_s[pl.ds(t0, 8), :]
            bu8 = bu_s[pl.ds(t0, 8), :]
            c8 = c_s[pl.ds(t0, 8), :]
            dd8 = dd_s[pl.ds(t0, 8), :]
            r8 = r_s[pl.ds(t0, 8), :]
            y8 = dd8
            for j in range(8):
                mask = eidx == r8[j:j + 1, :]
                aj = a8[j:j + 1, :]
                buj = bu8[j:j + 1, :]
                h = jnp.where(mask, aj * h + buj, h)
                hr = jnp.sum(jnp.where(mask, h, 0.0), axis=0, keepdims=True)
                y8 = y8 + jnp.where(sidx == j, c8[j:j + 1, :] * hr, 0.0)
            y_ref[pl.ds(t0, 8), :] = y8
            return h

        jax.lax.fori_loop(0, T // 8, body, jnp.zeros((E, S), jnp.float32))


def _out_kernel(y_ref, r_ref, xm_ref, wout_ref, ssm_ref, mlp_ref,
                w1_ref, w2_ref, o_ref):
    f32 = jnp.float32
    r = r_ref[...]
    y = y_ref[...]
    out = jnp.zeros((CHUNK, D), f32)
    for e in range(E):
        ye = jnp.where(r == e, y, 0.0)
        out = out + jnp.dot(ye, wout_ref[e], preferred_element_type=f32)
    xm2 = xm_ref[...] + ssm_ref[...] * out
    ms = jnp.mean(xm2 * xm2, axis=1, keepdims=True)
    xn2 = xm2 * jax.lax.rsqrt(ms + 1e-6)
    hmid = jnp.dot(xn2, w1_ref[...], preferred_element_type=f32)
    hmid = jnp.square(jnp.maximum(hmid, 0.0))
    mlp = jnp.dot(hmid, w2_ref[...], preferred_element_type=f32)
    o_ref[...] = xm2 + mlp_ref[...] * mlp


def kernel(x, x0, token_ids, W_in, W_sel_in, W_sel_out, W_out, d_param,
           resid_mix, ssm_scale, mlp_scale, W_mlp1, W_mlp2):
    f32 = jnp.float32
    x2 = x.reshape(T, D)
    x02 = x0.reshape(T, D)
    tid = token_ids.reshape(T, 1)

    full = lambda shape: pl.BlockSpec(shape, lambda i: tuple(0 for _ in shape))
    chunk = lambda shape: pl.BlockSpec(shape, lambda i: (i,) + tuple(0 for _ in shape[1:]))

    xm, r, y = pl.pallas_call(
        _gates_kernel,
        grid=(NCHUNK,),
        in_specs=[
            chunk((CHUNK, D)), chunk((CHUNK, D)), chunk((CHUNK, 1)),
            full((E, D, S)), full((E, D, H)), full((E, H, 4 * S)),
            full((E, S)), full((2, D)),
        ],
        out_specs=[
            chunk((CHUNK, D)), chunk((CHUNK, 1)),
            pl.BlockSpec((T, S), lambda i: (0, 0)),
        ],
        out_shape=[
            jax.ShapeDtypeStruct((T, D), f32),
            jax.ShapeDtypeStruct((T, 1), jnp.int32),
            jax.ShapeDtypeStruct((T, S), f32),
        ],
        scratch_shapes=[
            pltpu.VMEM((T, S), jnp.float32),
            pltpu.VMEM((T, S), jnp.float32),
            pltpu.VMEM((T, S), jnp.float32),
            pltpu.VMEM((T, S), jnp.float32),
            pltpu.VMEM((T, 1), jnp.int32),
        ],
    )(x2, x02, tid, W_in, W_sel_in, W_sel_out, d_param, resid_mix)

    o = pl.pallas_call(
        _out_kernel,
        grid=(NCHUNK,),
        in_specs=[
            chunk((CHUNK, S)), chunk((CHUNK, 1)), chunk((CHUNK, D)),
            full((E, S, D)), full((1, D)), full((1, D)),
            full((D, 4 * D)), full((4 * D, D)),
        ],
        out_specs=chunk((CHUNK, D)),
        out_shape=jax.ShapeDtypeStruct((T, D), f32),
    )(y, r, xm, W_out, ssm_scale.reshape(1, D), mlp_scale.reshape(1, D),
      W_mlp1, W_mlp2)

    return o.reshape(1, T, D)


# speedup vs baseline: 1.1230x; 1.1170x over previous
import jax
import jax.numpy as jnp
from jax.experimental import pallas as pl
from jax.experimental.pallas import tpu as pltpu

T = 2048
D = 768
E = 8
S = 128
H = 128
CHUNK = 512
NCHUNK = T // CHUNK


def _routes_from_tokens(tid):
    x = tid
    x = x ^ jax.lax.shift_right_logical(x, 16)
    x = x * jnp.int32(-2048144789)
    x = x ^ jax.lax.shift_right_logical(x, 13)
    x = x * jnp.int32(-1028477387)
    x = x ^ jax.lax.shift_right_logical(x, 16)
    return x & jnp.int32(E - 1)


def _gates_kernel(x_ref, x0_ref, tid_ref, win_ref, wsi_ref, wso_ref, dp_ref,
                  rm_ref, xm_ref, r_ref, a_ref, bu_ref, c_ref, dd_ref):
    rm = rm_ref[...]
    xm = rm[0:1, :] * x_ref[...] + rm[1:2, :] * x0_ref[...]
    xm_ref[...] = xm
    ms = jnp.mean(xm * xm, axis=1, keepdims=True)
    xn = xm * jax.lax.rsqrt(ms + 1e-6)

    r = _routes_from_tokens(tid_ref[...])
    r_ref[...] = r

    f32 = jnp.float32
    u = jnp.zeros((CHUNK, S), f32)
    selz = jnp.zeros((CHUNK, H), f32)
    for e in range(E):
        xe = jnp.where(r == e, xn, 0.0)
        u = u + jnp.dot(xe, win_ref[e], preferred_element_type=f32)
        selz = selz + jnp.dot(xe, wsi_ref[e], preferred_element_type=f32)
    sel = selz * jax.nn.sigmoid(selz)
    so = jnp.zeros((CHUNK, 4 * S), f32)
    dp = jnp.zeros((CHUNK, S), f32)
    for e in range(E):
        se = jnp.where(r == e, sel, 0.0)
        so = so + jnp.dot(se, wso_ref[e], preferred_element_type=f32)
        dp = dp + (r == e).astype(f32) * dp_ref[e:e + 1, :]
    a = jax.nn.sigmoid(so[:, 0:S])
    b = jnp.tanh(so[:, S:2 * S])
    c = jnp.tanh(so[:, 2 * S:3 * S])
    dg = jax.nn.sigmoid(so[:, 3 * S:4 * S])
    a_ref[...] = a
    bu_ref[...] = b * u
    c_ref[...] = c
    dd_ref[...] = dp * dg * u


def _scan_kernel(r_ref, a_ref, bu_ref, c_ref, dd_ref, y_ref):
    eidx = jax.lax.broadcasted_iota(jnp.int32, (E, 1), 0)

    def body(blk, h):
        t0 = pl.multiple_of(blk * 8, 8)
        a8 = a_ref[pl.ds(t0, 8), :]
        bu8 = bu_ref[pl.ds(t0, 8), :]
        c8 = c_ref[pl.ds(t0, 8), :]
        dd8 = dd_ref[pl.ds(t0, 8), :]
        y8 = dd8
        for j in range(8):
            rt = r_ref[t0 + j]
            mask = eidx == rt
            aj = a8[j:j + 1, :]
            buj = bu8[j:j + 1, :]
            h = jnp.where(mask, aj * h + buj, h)
            hr = jnp.sum(jnp.where(mask, h, 0.0), axis=0, keepdims=True)
            yj = c8[j:j + 1, :] * hr
            y8 = y8 + jnp.where(jax.lax.broadcasted_iota(jnp.int32, (8, 1), 0) == j,
                                yj, 0.0)
        y_ref[pl.ds(t0, 8), :] = y8
        return h

    jax.lax.fori_loop(0, T // 8, body, jnp.zeros((E, S), jnp.float32))


def _out_kernel(y_ref, r_ref, xm_ref, wout_ref, ssm_ref, mlp_ref,
                w1_ref, w2_ref, o_ref):
    f32 = jnp.float32
    r = r_ref[...]
    y = y_ref[...]
    out = jnp.zeros((CHUNK, D), f32)
    for e in range(E):
        ye = jnp.where(r == e, y, 0.0)
        out = out + jnp.dot(ye, wout_ref[e], preferred_element_type=f32)
    xm2 = xm_ref[...] + ssm_ref[...] * out
    ms = jnp.mean(xm2 * xm2, axis=1, keepdims=True)
    xn2 = xm2 * jax.lax.rsqrt(ms + 1e-6)
    hmid = jnp.dot(xn2, w1_ref[...], preferred_element_type=f32)
    hmid = jnp.square(jnp.maximum(hmid, 0.0))
    mlp = jnp.dot(hmid, w2_ref[...], preferred_element_type=f32)
    o_ref[...] = xm2 + mlp_ref[...] * mlp


def kernel(x, x0, token_ids, W_in, W_sel_in, W_sel_out, W_out, d_param,
           resid_mix, ssm_scale, mlp_scale, W_mlp1, W_mlp2):
    f32 = jnp.float32
    x2 = x.reshape(T, D)
    x02 = x0.reshape(T, D)
    tid = token_ids.reshape(T, 1)

    full = lambda shape: pl.BlockSpec(shape, lambda i: tuple(0 for _ in shape))
    chunk = lambda shape: pl.BlockSpec(shape, lambda i: (i,) + tuple(0 for _ in shape[1:]))

    xm, r, a, bu, c, dd = pl.pallas_call(
        _gates_kernel,
        grid=(NCHUNK,),
        in_specs=[
            chunk((CHUNK, D)), chunk((CHUNK, D)), chunk((CHUNK, 1)),
            full((E, D, S)), full((E, D, H)), full((E, H, 4 * S)),
            full((E, S)), full((2, D)),
        ],
        out_specs=[
            chunk((CHUNK, D)), chunk((CHUNK, 1)), chunk((CHUNK, S)),
            chunk((CHUNK, S)), chunk((CHUNK, S)), chunk((CHUNK, S)),
        ],
        out_shape=[
            jax.ShapeDtypeStruct((T, D), f32),
            jax.ShapeDtypeStruct((T, 1), jnp.int32),
            jax.ShapeDtypeStruct((T, S), f32),
            jax.ShapeDtypeStruct((T, S), f32),
            jax.ShapeDtypeStruct((T, S), f32),
            jax.ShapeDtypeStruct((T, S), f32),
        ],
    )(x2, x02, tid, W_in, W_sel_in, W_sel_out, d_param, resid_mix)

    y = pl.pallas_call(
        _scan_kernel,
        grid_spec=pltpu.PrefetchScalarGridSpec(
            num_scalar_prefetch=1,
            grid=(1,),
            in_specs=[
                pl.BlockSpec((T, S), lambda i, s: (0, 0)),
                pl.BlockSpec((T, S), lambda i, s: (0, 0)),
                pl.BlockSpec((T, S), lambda i, s: (0, 0)),
                pl.BlockSpec((T, S), lambda i, s: (0, 0)),
            ],
            out_specs=pl.BlockSpec((T, S), lambda i, s: (0, 0)),
        ),
        out_shape=jax.ShapeDtypeStruct((T, S), f32),
    )(r.reshape(T), a, bu, c, dd)

    o = pl.pallas_call(
        _out_kernel,
        grid=(NCHUNK,),
        in_specs=[
            chunk((CHUNK, S)), chunk((CHUNK, 1)), chunk((CHUNK, D)),
            full((E, S, D)), full((1, D)), full((1, D)),
            full((D, 4 * D)), full((4 * D, D)),
        ],
        out_specs=chunk((CHUNK, D)),
        out_shape=jax.ShapeDtypeStruct((T, D), f32),
    )(y, r, xm, W_out, ssm_scale.reshape(1, D), mlp_scale.reshape(1, D),
      W_mlp1, W_mlp2)

    return o.reshape(1, T, D)


# speedup vs baseline: 1.2861x; 1.1452x over previous
import jax
import jax.numpy as jnp
from jax.experimental import pallas as pl
from jax.experimental.pallas import tpu as pltpu

T = 2048
D = 768
E = 8
S = 128
H = 128
CHUNK = 512
NCHUNK = T // CHUNK


def _routes_from_tokens(tid):
    x = tid
    x = x ^ jax.lax.shift_right_logical(x, 16)
    x = x * jnp.int32(-2048144789)
    x = x ^ jax.lax.shift_right_logical(x, 13)
    x = x * jnp.int32(-1028477387)
    x = x ^ jax.lax.shift_right_logical(x, 16)
    return x & jnp.int32(E - 1)


def _gates_kernel(x_ref, x0_ref, tid_ref, win_ref, wsi_ref, wso_ref, dp_ref,
                  rm_ref, xm_ref, r_ref, a_ref, bu_ref, c_ref, dd_ref):
    rm = rm_ref[...]
    xm = rm[0:1, :] * x_ref[...] + rm[1:2, :] * x0_ref[...]
    xm_ref[...] = xm
    ms = jnp.mean(xm * xm, axis=1, keepdims=True)
    xn = xm * jax.lax.rsqrt(ms + 1e-6)

    r = _routes_from_tokens(tid_ref[...])
    r_ref[...] = r

    f32 = jnp.float32
    xc = jnp.concatenate([jnp.where(r == e, xn, 0.0) for e in range(E)], axis=1)
    u = jnp.dot(xc, win_ref[...], preferred_element_type=f32)
    selz = jnp.dot(xc, wsi_ref[...], preferred_element_type=f32)
    sel = selz * jax.nn.sigmoid(selz)
    sc = jnp.concatenate([jnp.where(r == e, sel, 0.0) for e in range(E)], axis=1)
    so = jnp.dot(sc, wso_ref[...], preferred_element_type=f32)
    dp = jnp.zeros((CHUNK, S), f32)
    for e in range(E):
        dp = dp + (r == e).astype(f32) * dp_ref[e:e + 1, :]
    a = jax.nn.sigmoid(so[:, 0:S])
    b = jnp.tanh(so[:, S:2 * S])
    c = jnp.tanh(so[:, 2 * S:3 * S])
    dg = jax.nn.sigmoid(so[:, 3 * S:4 * S])
    a_ref[...] = a
    bu_ref[...] = b * u
    c_ref[...] = c
    dd_ref[...] = dp * dg * u


def _scan_kernel(r_ref, a_ref, bu_ref, c_ref, dd_ref, y_ref):
    eidx = jax.lax.broadcasted_iota(jnp.int32, (E, 1), 0)

    def body(blk, h):
        t0 = pl.multiple_of(blk * 8, 8)
        a8 = a_ref[pl.ds(t0, 8), :]
        bu8 = bu_ref[pl.ds(t0, 8), :]
        c8 = c_ref[pl.ds(t0, 8), :]
        dd8 = dd_ref[pl.ds(t0, 8), :]
        y8 = dd8
        for j in range(8):
            rt = r_ref[t0 + j]
            mask = eidx == rt
            aj = a8[j:j + 1, :]
            buj = bu8[j:j + 1, :]
            h = jnp.where(mask, aj * h + buj, h)
            hr = jnp.sum(jnp.where(mask, h, 0.0), axis=0, keepdims=True)
            yj = c8[j:j + 1, :] * hr
            y8 = y8 + jnp.where(jax.lax.broadcasted_iota(jnp.int32, (8, 1), 0) == j,
                                yj, 0.0)
        y_ref[pl.ds(t0, 8), :] = y8
        return h

    jax.lax.fori_loop(0, T // 8, body, jnp.zeros((E, S), jnp.float32))


def _out_kernel(y_ref, r_ref, xm_ref, wout_ref, ssm_ref, mlp_ref,
                w1_ref, w2_ref, o_ref):
    f32 = jnp.float32
    r = r_ref[...]
    y = y_ref[...]
    yc = jnp.concatenate([jnp.where(r == e, y, 0.0) for e in range(E)], axis=1)
    out = jnp.dot(yc, wout_ref[...], preferred_element_type=f32)
    xm2 = xm_ref[...] + ssm_ref[...] * out
    ms = jnp.mean(xm2 * xm2, axis=1, keepdims=True)
    xn2 = xm2 * jax.lax.rsqrt(ms + 1e-6)
    hmid = jnp.dot(xn2, w1_ref[...], preferred_element_type=f32)
    hmid = jnp.square(jnp.maximum(hmid, 0.0))
    mlp = jnp.dot(hmid, w2_ref[...], preferred_element_type=f32)
    o_ref[...] = xm2 + mlp_ref[...] * mlp


def kernel(x, x0, token_ids, W_in, W_sel_in, W_sel_out, W_out, d_param,
           resid_mix, ssm_scale, mlp_scale, W_mlp1, W_mlp2):
    f32 = jnp.float32
    x2 = x.reshape(T, D)
    x02 = x0.reshape(T, D)
    tid = token_ids.reshape(T, 1)

    full = lambda shape: pl.BlockSpec(shape, lambda i: tuple(0 for _ in shape))
    chunk = lambda shape: pl.BlockSpec(shape, lambda i: (i,) + tuple(0 for _ in shape[1:]))

    xm, r, a, bu, c, dd = pl.pallas_call(
        _gates_kernel,
        grid=(NCHUNK,),
        in_specs=[
            chunk((CHUNK, D)), chunk((CHUNK, D)), chunk((CHUNK, 1)),
            full((E * D, S)), full((E * D, H)), full((E * H, 4 * S)),
            full((E, S)), full((2, D)),
        ],
        out_specs=[
            chunk((CHUNK, D)), chunk((CHUNK, 1)), chunk((CHUNK, S)),
            chunk((CHUNK, S)), chunk((CHUNK, S)), chunk((CHUNK, S)),
        ],
        out_shape=[
            jax.ShapeDtypeStruct((T, D), f32),
            jax.ShapeDtypeStruct((T, 1), jnp.int32),
            jax.ShapeDtypeStruct((T, S), f32),
            jax.ShapeDtypeStruct((T, S), f32),
            jax.ShapeDtypeStruct((T, S), f32),
            jax.ShapeDtypeStruct((T, S), f32),
        ],
    )(x2, x02, tid, W_in.reshape(E * D, S), W_sel_in.reshape(E * D, H),
      W_sel_out.reshape(E * H, 4 * S), d_param, resid_mix)

    y = pl.pallas_call(
        _scan_kernel,
        grid_spec=pltpu.PrefetchScalarGridSpec(
            num_scalar_prefetch=1,
            grid=(1,),
            in_specs=[
                pl.BlockSpec((T, S), lambda i, s: (0, 0)),
                pl.BlockSpec((T, S), lambda i, s: (0, 0)),
                pl.BlockSpec((T, S), lambda i, s: (0, 0)),
                pl.BlockSpec((T, S), lambda i, s: (0, 0)),
            ],
            out_specs=pl.BlockSpec((T, S), lambda i, s: (0, 0)),
        ),
        out_shape=jax.ShapeDtypeStruct((T, S), f32),
    )(r.reshape(T), a, bu, c, dd)

    o = pl.pallas_call(
        _out_kernel,
        grid=(NCHUNK,),
        in_specs=[
            chunk((CHUNK, S)), chunk((CHUNK, 1)), chunk((CHUNK, D)),
            full((E * S, D)), full((1, D)), full((1, D)),
            full((D, 4 * D)), full((4 * D, D)),
        ],
        out_specs=chunk((CHUNK, D)),
        out_shape=jax.ShapeDtypeStruct((T, D), f32),
    )(y, r, xm, W_out.reshape(E * S, D), ssm_scale.reshape(1, D),
      mlp_scale.reshape(1, D), W_mlp1, W_mlp2)

    return o.reshape(1, T, D)
